# Initial kernel scaffold; baseline (speedup 1.0000x reference)
import jax
import jax.numpy as jnp
from jax.experimental import pallas as pl


def kernel(x, embeddings, edge_index, Wq, Wk, v_w, fc_w, fc_b):
    raise NotImplementedError("write your pallas kernel here")


# speedup vs baseline: 5.9139x; 5.9139x over previous
import jax
import jax.numpy as jnp
from jax import lax
from jax.experimental import pallas as pl
from jax.experimental.pallas import tpu as pltpu
from jax.experimental.pallas import tpu_sc as plsc

N = 10000
E = 320000
F = 128
D = 16
OUT = 128
NC = 2
NS = 16
NW = NC * NS
EPW = E // NW
NGRP = EPW // 16
CH = 80
NCH = EPW // CH
NZCH = N // CH

_MESH = plsc.VectorSubcoreMesh(
    core_axis_name="c", subcore_axis_name="s", num_cores=NC, num_subcores=NS)
_SC_PARAMS = pltpu.CompilerParams(needs_layout_passes=False)


def _qk_body(embT_ref, wq_ref, wk_ref, qT_ref, kT_ref):
    embT = embT_ref[...]
    qT_ref[...] = jnp.dot(wq_ref[...], embT, preferred_element_type=jnp.float32)
    kT_ref[...] = jnp.dot(wk_ref[...], embT, preferred_element_type=jnp.float32)


def _qk(embT, Wq, Wk):
    return pl.pallas_call(
        _qk_body,
        out_shape=(jax.ShapeDtypeStruct((D, N), jnp.float32),
                   jax.ShapeDtypeStruct((D, N), jnp.float32)),
    )(embT, Wq, Wk)


def _score_body(qT_h, kT_h, src_h, dst_h, vb_h, w_h,
                colq, colk, srcv, dstv, acc, vbuf):
    c = lax.axis_index("c")
    s = lax.axis_index("s")
    wid = s * NC + c
    base = wid * EPW
    pltpu.sync_copy(src_h.at[pl.ds(base, EPW)], srcv)
    pltpu.sync_copy(dst_h.at[pl.ds(base, EPW)], dstv)
    pltpu.sync_copy(vb_h, vbuf)

    for d in range(D):
        pltpu.sync_copy(qT_h.at[pl.ds(d * N, N)], colq)
        pltpu.sync_copy(kT_h.at[pl.ds(d * N, N)], colk)
        vrow = vbuf[d]

        def d_body(g, carry):
            off = g * 16
            s16 = srcv[pl.ds(off, 16)]
            d16 = dstv[pl.ds(off, 16)]
            qv = plsc.load_gather(colq, [s16])
            kv = plsc.load_gather(colk, [d16])
            z = qv + kv
            e2 = jnp.exp(z + z)
            t = 1.0 - 2.0 / (e2 + 1.0)
            if d == 0:
                acc[pl.ds(off, 16)] = vrow * t
            else:
                acc[pl.ds(off, 16)] = acc[pl.ds(off, 16)] + vrow * t
            return carry
        lax.fori_loop(0, NGRP, d_body, 0)

    def fin_body(g, carry):
        off = g * 16
        acc[pl.ds(off, 16)] = jnp.exp(acc[pl.ds(off, 16)])
        return carry
    lax.fori_loop(0, NGRP, fin_body, 0)

    pltpu.sync_copy(acc, w_h.at[pl.ds(base, EPW)])


def _score(qT, kT, src, dst, vb):
    fn = pl.kernel(
        _score_body,
        out_type=jax.ShapeDtypeStruct((E,), jnp.float32),
        mesh=_MESH,
        compiler_params=_SC_PARAMS,
        scratch_types=[
            pltpu.VMEM((N,), jnp.float32),
            pltpu.VMEM((N,), jnp.float32),
            pltpu.VMEM((EPW,), jnp.int32),
            pltpu.VMEM((EPW,), jnp.int32),
            pltpu.VMEM((EPW,), jnp.float32),
            pltpu.VMEM((D, 16), jnp.float32),
        ],
    )
    return fn(qT, kT, src, dst, vb)


def _agg_body(x_h, src_h, dst_h, w_h, aggp_h, denp_h,
              sidx, wchunk, didx, zbuf, xbuf, sbuf, agg_sh, den_sh):
    c = lax.axis_index("c")
    s = lax.axis_index("s")
    wid = s * NC + c
    base = wid * EPW

    for r in range(CH):
        for h in range(F // 16):
            xbuf[r, pl.ds(h * 16, 16)] = jnp.zeros((16,), jnp.float32)
    for h in range(CH // 16):
        zbuf[pl.ds(h * 16, 16)] = jnp.zeros((16,), jnp.float32)
    for i in range((NZCH + NS - 1) // NS):
        j = s + i * NS

        @pl.when(j < NZCH)
        def _():
            pltpu.sync_copy(xbuf, agg_sh.at[pl.ds(j * CH, CH)])
            pltpu.sync_copy(zbuf, den_sh.at[pl.ds(j * CH, CH)])
    plsc.subcore_barrier()

    def chunk_body(j, carry):
        pltpu.sync_copy(w_h.at[pl.ds(base + j * CH, CH)], wchunk)
        pltpu.sync_copy(dst_h.at[pl.ds(base + j * CH, CH)], didx)
        pltpu.sync_copy(src_h.at[pl.ds(base + j * CH, CH)], sidx)
        pltpu.sync_copy(x_h.at[sidx], xbuf)

        def e_body(e, carry2):
            a16 = plsc.load_gather(
                wchunk, [jnp.broadcast_to(e, (16,)).astype(jnp.int32)])
            for h in range(F // 16):
                sbuf[e, pl.ds(h * 16, 16)] = a16 * xbuf[e, pl.ds(h * 16, 16)]
            return carry2
        lax.fori_loop(0, CH, e_body, 0)
        pltpu.sync_copy(sbuf, agg_sh.at[didx], add=True)
        pltpu.sync_copy(wchunk, den_sh.at[didx], add=True)
        return carry
    lax.fori_loop(0, NCH, chunk_body, 0)

    plsc.subcore_barrier()
    for i in range((NZCH + NS - 1) // NS):
        j = s + i * NS

        @pl.when(j < NZCH)
        def _():
            pltpu.sync_copy(agg_sh.at[pl.ds(j * CH, CH)],
                            aggp_h.at[c, pl.ds(j * CH, CH)])
            pltpu.sync_copy(den_sh.at[pl.ds(j * CH, CH)], zbuf)
            pltpu.sync_copy(zbuf, denp_h.at[pl.ds(c * N + j * CH, CH)])


def _agg(x2d, src, dst, w):
    fn = pl.kernel(
        _agg_body,
        out_type=(jax.ShapeDtypeStruct((NC, N, F), jnp.float32),
                  jax.ShapeDtypeStruct((NC * N,), jnp.float32)),
        mesh=_MESH,
        compiler_params=_SC_PARAMS,
        scratch_types=[
            pltpu.VMEM((CH,), jnp.int32),
            pltpu.VMEM((CH,), jnp.float32),
            pltpu.VMEM((CH,), jnp.int32),
            pltpu.VMEM((CH,), jnp.float32),
            pltpu.VMEM((CH, F), jnp.float32),
            pltpu.VMEM((CH, F), jnp.float32),
            pltpu.VMEM_SHARED((N, F), jnp.float32),
            pltpu.VMEM_SHARED((N,), jnp.float32),
        ],
    )
    return fn(x2d, src, dst, w)


def _fc_body(x_ref, a0_ref, a1_ref, d0_ref, d1_ref, w1_ref, w2_ref, b_ref,
             o_ref):
    inv = 1.0 / (d0_ref[...] + d1_ref[...] + 1e-8)
    ag = (a0_ref[...] + a1_ref[...]) * inv
    acc = jnp.dot(x_ref[...], w1_ref[...], preferred_element_type=jnp.float32)
    acc = acc + jnp.dot(ag, w2_ref[...], preferred_element_type=jnp.float32)
    o_ref[...] = jnp.maximum(acc + b_ref[...], 0.0)


def _fc(x2d, a0, a1, d0, d1, w1T, w2T, b2d):
    BLK = 1000
    return pl.pallas_call(
        _fc_body,
        grid=(N // BLK,),
        in_specs=[
            pl.BlockSpec((BLK, F), lambda i: (i, 0)),
            pl.BlockSpec((BLK, F), lambda i: (i, 0)),
            pl.BlockSpec((BLK, F), lambda i: (i, 0)),
            pl.BlockSpec((BLK, 1), lambda i: (i, 0)),
            pl.BlockSpec((BLK, 1), lambda i: (i, 0)),
            pl.BlockSpec((F, OUT), lambda i: (0, 0)),
            pl.BlockSpec((F, OUT), lambda i: (0, 0)),
            pl.BlockSpec((1, OUT), lambda i: (0, 0)),
        ],
        out_specs=pl.BlockSpec((BLK, OUT), lambda i: (i, 0)),
        out_shape=jax.ShapeDtypeStruct((N, OUT), jnp.float32),
    )(x2d, a0, a1, d0, d1, w1T, w2T, b2d)


def kernel(x, embeddings, edge_index, Wq, Wk, v_w, fc_w, fc_b):
    x2d = x[0]
    embT = embeddings.T
    src = edge_index[0]
    dst = edge_index[1]
    vb = jnp.broadcast_to(v_w.reshape(D, 1), (D, 16))

    qT, kT = _qk(embT, Wq, Wk)
    w = _score(qT.reshape(D * N), kT.reshape(D * N), src, dst, vb)

    aggp, denp = _agg(x2d, src, dst, w)
    denp2 = denp.reshape(NC, N)

    out2d = _fc(x2d, aggp[0], aggp[1],
                denp2[0].reshape(N, 1), denp2[1].reshape(N, 1),
                fc_w[:, :F].T, fc_w[:, F:].T, fc_b.reshape(1, OUT))
    return out2d.reshape(1, N, OUT)


# speedup vs baseline: 11.3666x; 1.9220x over previous
import jax
import jax.numpy as jnp
from jax import lax
from jax.experimental import pallas as pl
from jax.experimental.pallas import tpu as pltpu
from jax.experimental.pallas import tpu_sc as plsc

N = 10000
E = 320000
F = 128
D = 16
OUT = 128
NC = 2
NS = 16
NW = NC * NS
EPW = E // NW
NGRP = EPW // 16
CH = 80
NCH = EPW // CH
NZCH = N // CH

_MESH = plsc.VectorSubcoreMesh(
    core_axis_name="c", subcore_axis_name="s", num_cores=NC, num_subcores=NS)
_SC_PARAMS = pltpu.CompilerParams(needs_layout_passes=False)


def _qk_body(embT_ref, wq_ref, wk_ref, qT_ref, kT_ref):
    embT = embT_ref[...]
    qT_ref[...] = jnp.dot(wq_ref[...], embT, preferred_element_type=jnp.float32)
    kT_ref[...] = jnp.dot(wk_ref[...], embT, preferred_element_type=jnp.float32)


def _qk(embT, Wq, Wk):
    return pl.pallas_call(
        _qk_body,
        out_shape=(jax.ShapeDtypeStruct((D, N), jnp.float32),
                   jax.ShapeDtypeStruct((D, N), jnp.float32)),
    )(embT, Wq, Wk)


def _score_body(qT_h, kT_h, src_h, dst_h, vb_h, w_h,
                colq, colk, srcv, dstv, acc, vbuf):
    c = lax.axis_index("c")
    s = lax.axis_index("s")
    wid = s * NC + c
    base = wid * EPW
    pltpu.sync_copy(src_h.at[pl.ds(base, EPW)], srcv)
    pltpu.sync_copy(dst_h.at[pl.ds(base, EPW)], dstv)
    pltpu.sync_copy(vb_h, vbuf)

    for d in range(D):
        pltpu.sync_copy(qT_h.at[pl.ds(d * N, N)], colq)
        pltpu.sync_copy(kT_h.at[pl.ds(d * N, N)], colk)
        vrow = vbuf[d]

        def d_body(g, carry):
            off = g * 16
            s16 = srcv[pl.ds(off, 16)]
            d16 = dstv[pl.ds(off, 16)]
            qv = plsc.load_gather(colq, [s16])
            kv = plsc.load_gather(colk, [d16])
            z = qv + kv
            e2 = jnp.exp(z + z)
            t = 1.0 - 2.0 / (e2 + 1.0)
            if d == 0:
                acc[pl.ds(off, 16)] = vrow * t
            else:
                acc[pl.ds(off, 16)] = acc[pl.ds(off, 16)] + vrow * t
            return carry
        lax.fori_loop(0, NGRP, d_body, 0)

    def fin_body(g, carry):
        off = g * 16
        acc[pl.ds(off, 16)] = jnp.exp(acc[pl.ds(off, 16)])
        return carry
    lax.fori_loop(0, NGRP, fin_body, 0)

    pltpu.sync_copy(acc, w_h.at[pl.ds(base, EPW)])


def _score(qT, kT, src, dst, vb):
    fn = pl.kernel(
        _score_body,
        out_type=jax.ShapeDtypeStruct((E,), jnp.float32),
        mesh=_MESH,
        compiler_params=_SC_PARAMS,
        scratch_types=[
            pltpu.VMEM((N,), jnp.float32),
            pltpu.VMEM((N,), jnp.float32),
            pltpu.VMEM((EPW,), jnp.int32),
            pltpu.VMEM((EPW,), jnp.int32),
            pltpu.VMEM((EPW,), jnp.float32),
            pltpu.VMEM((D, 16), jnp.float32),
        ],
    )
    return fn(qT, kT, src, dst, vb)


def _agg_body(x_h, src_h, dst_h, w_h, aggp_h, denp_h,
              mb_s, mb_d, mb_w, zbuf, xbufs, agg_sh, den_sh,
              sem_g, sem_s, sem_m):
    c = lax.axis_index("c")
    s = lax.axis_index("s")
    wid = s * NC + c
    base = wid * EPW

    def _splat(v):
        return jnp.broadcast_to(v, (16,)).astype(jnp.int32)

    for r in range(CH):
        for h in range(F // 16):
            xbufs[0, r, pl.ds(h * 16, 16)] = jnp.zeros((16,), jnp.float32)
    for h in range(CH // 16):
        zbuf[pl.ds(h * 16, 16)] = jnp.zeros((16,), jnp.float32)
    for i in range((NZCH + NS - 1) // NS):
        j = s + i * NS

        @pl.when(j < NZCH)
        def _():
            pltpu.sync_copy(xbufs.at[0], agg_sh.at[pl.ds(j * CH, CH)])
            pltpu.sync_copy(zbuf, den_sh.at[pl.ds(j * CH, CH)])
    plsc.subcore_barrier()

    def _meta_slices(j):
        return (src_h.at[pl.ds(base + j * CH, CH)],
                dst_h.at[pl.ds(base + j * CH, CH)],
                w_h.at[pl.ds(base + j * CH, CH)])

    s0, d0, w0 = _meta_slices(0)
    pltpu.sync_copy(s0, mb_s.at[0])
    pltpu.sync_copy(d0, mb_d.at[0])
    pltpu.sync_copy(w0, mb_w.at[0])
    s1, d1, w1 = _meta_slices(1)
    pltpu.async_copy(s1, mb_s.at[1], sem_m)
    pltpu.async_copy(d1, mb_d.at[1], sem_m)
    pltpu.async_copy(w1, mb_w.at[1], sem_m)
    pltpu.async_copy(x_h.at[mb_s.at[0]], xbufs.at[0], sem_g)

    def chunk_body(j, carry):
        m0 = j % 3
        m1 = (j + 1) % 3
        m2 = (j + 2) % 3

        pltpu.make_async_copy(x_h.at[mb_s.at[m0]], xbufs.at[m0], sem_g).wait()

        @plsc.parallel_loop(0, CH, unroll=2)
        def _(e):
            a16 = plsc.load_gather(mb_w, [_splat(m0), _splat(e)])
            for h in range(F // 16):
                xbufs[m0, e, pl.ds(h * 16, 16)] = (
                    a16 * xbufs[m0, e, pl.ds(h * 16, 16)])

        @pl.when(j > 0)
        def _():
            pltpu.make_async_copy(
                xbufs.at[m2], agg_sh.at[mb_d.at[m2]], sem_s).wait()
            pltpu.make_async_copy(
                mb_w.at[m2], den_sh.at[mb_d.at[m2]], sem_s).wait()

        pltpu.async_copy(xbufs.at[m0], agg_sh.at[mb_d.at[m0]], sem_s,
                         add=True)
        pltpu.async_copy(mb_w.at[m0], den_sh.at[mb_d.at[m0]], sem_s,
                         add=True)

        @pl.when(j + 1 < NCH)
        def _():
            sj, dj, wj = _meta_slices(j + 1)
            pltpu.make_async_copy(sj, mb_s.at[m1], sem_m).wait()
            pltpu.make_async_copy(dj, mb_d.at[m1], sem_m).wait()
            pltpu.make_async_copy(wj, mb_w.at[m1], sem_m).wait()

        @pl.when(j + 2 < NCH)
        def _():
            sj, dj, wj = _meta_slices(j + 2)
            pltpu.async_copy(sj, mb_s.at[m2], sem_m)
            pltpu.async_copy(dj, mb_d.at[m2], sem_m)
            pltpu.async_copy(wj, mb_w.at[m2], sem_m)

        @pl.when(j + 1 < NCH)
        def _():
            pltpu.async_copy(x_h.at[mb_s.at[m1]], xbufs.at[m1], sem_g)
        return carry
    lax.fori_loop(0, NCH, chunk_body, 0)

    ml = (NCH - 1) % 3
    pltpu.make_async_copy(xbufs.at[ml], agg_sh.at[mb_d.at[ml]], sem_s).wait()
    pltpu.make_async_copy(mb_w.at[ml], den_sh.at[mb_d.at[ml]], sem_s).wait()

    plsc.subcore_barrier()
    for i in range((NZCH + NS - 1) // NS):
        j = s + i * NS

        @pl.when(j < NZCH)
        def _():
            pltpu.sync_copy(agg_sh.at[pl.ds(j * CH, CH)],
                            aggp_h.at[c, pl.ds(j * CH, CH)])
            pltpu.sync_copy(den_sh.at[pl.ds(j * CH, CH)], zbuf)
            pltpu.sync_copy(zbuf, denp_h.at[pl.ds(c * N + j * CH, CH)])


def _agg(x2d, src, dst, w):
    fn = pl.kernel(
        _agg_body,
        out_type=(jax.ShapeDtypeStruct((NC, N, F), jnp.float32),
                  jax.ShapeDtypeStruct((NC * N,), jnp.float32)),
        mesh=_MESH,
        compiler_params=_SC_PARAMS,
        scratch_types=[
            pltpu.VMEM((3, CH), jnp.int32),
            pltpu.VMEM((3, CH), jnp.int32),
            pltpu.VMEM((3, CH), jnp.float32),
            pltpu.VMEM((CH,), jnp.float32),
            pltpu.VMEM((3, CH, F), jnp.float32),
            pltpu.VMEM_SHARED((N, F), jnp.float32),
            pltpu.VMEM_SHARED((N,), jnp.float32),
            pltpu.SemaphoreType.DMA,
            pltpu.SemaphoreType.DMA,
            pltpu.SemaphoreType.DMA,
        ],
    )
    return fn(x2d, src, dst, w)


def _fc_body(x_ref, a0_ref, a1_ref, d0_ref, d1_ref, w1_ref, w2_ref, b_ref,
             o_ref):
    inv = 1.0 / (d0_ref[...] + d1_ref[...] + 1e-8)
    ag = (a0_ref[...] + a1_ref[...]) * inv
    acc = jnp.dot(x_ref[...], w1_ref[...], preferred_element_type=jnp.float32)
    acc = acc + jnp.dot(ag, w2_ref[...], preferred_element_type=jnp.float32)
    o_ref[...] = jnp.maximum(acc + b_ref[...], 0.0)


def _fc(x2d, a0, a1, d0, d1, w1T, w2T, b2d):
    BLK = 1000
    return pl.pallas_call(
        _fc_body,
        grid=(N // BLK,),
        in_specs=[
            pl.BlockSpec((BLK, F), lambda i: (i, 0)),
            pl.BlockSpec((BLK, F), lambda i: (i, 0)),
            pl.BlockSpec((BLK, F), lambda i: (i, 0)),
            pl.BlockSpec((BLK, 1), lambda i: (i, 0)),
            pl.BlockSpec((BLK, 1), lambda i: (i, 0)),
            pl.BlockSpec((F, OUT), lambda i: (0, 0)),
            pl.BlockSpec((F, OUT), lambda i: (0, 0)),
            pl.BlockSpec((1, OUT), lambda i: (0, 0)),
        ],
        out_specs=pl.BlockSpec((BLK, OUT), lambda i: (i, 0)),
        out_shape=jax.ShapeDtypeStruct((N, OUT), jnp.float32),
    )(x2d, a0, a1, d0, d1, w1T, w2T, b2d)


def kernel(x, embeddings, edge_index, Wq, Wk, v_w, fc_w, fc_b):
    x2d = x[0]
    embT = embeddings.T
    src = edge_index[0]
    dst = edge_index[1]
    vb = jnp.broadcast_to(v_w.reshape(D, 1), (D, 16))

    qT, kT = _qk(embT, Wq, Wk)
    w = _score(qT.reshape(D * N), kT.reshape(D * N), src, dst, vb)

    aggp, denp = _agg(x2d, src, dst, w)
    denp2 = denp.reshape(NC, N)

    out2d = _fc(x2d, aggp[0], aggp[1],
                denp2[0].reshape(N, 1), denp2[1].reshape(N, 1),
                fc_w[:, :F].T, fc_w[:, F:].T, fc_b.reshape(1, OUT))
    return out2d.reshape(1, N, OUT)


# speedup vs baseline: 20.1278x; 1.7708x over previous
import jax
import jax.numpy as jnp
from jax import lax
from jax.experimental import pallas as pl
from jax.experimental.pallas import tpu as pltpu
from jax.experimental.pallas import tpu_sc as plsc

N = 10000
E = 320000
F = 128
D = 16
OUT = 128
NC = 2
NS = 16
NW = NC * NS
EPW = E // NW
NGRP = EPW // 16
CH = 80
NCH = EPW // CH
NZCH = N // CH

_MESH = plsc.VectorSubcoreMesh(
    core_axis_name="c", subcore_axis_name="s", num_cores=NC, num_subcores=NS)
_SC_PARAMS = pltpu.CompilerParams(needs_layout_passes=False)


def _qk_body(embT_ref, wq_ref, wk_ref, qT_ref, kT_ref):
    embT = embT_ref[...]
    qT_ref[...] = jnp.dot(wq_ref[...], embT, preferred_element_type=jnp.float32)
    kT_ref[...] = jnp.dot(wk_ref[...], embT, preferred_element_type=jnp.float32)


def _qk(embT, Wq, Wk):
    return pl.pallas_call(
        _qk_body,
        out_shape=(jax.ShapeDtypeStruct((D, N), jnp.float32),
                   jax.ShapeDtypeStruct((D, N), jnp.float32)),
    )(embT, Wq, Wk)


def _score_body(qT_h, kT_h, src_h, dst_h, vb_h, w_h,
                colq0, colk0, colq1, colk1, srcv, dstv, acc, vbuf, sem_c):
    c = lax.axis_index("c")
    s = lax.axis_index("s")
    wid = s * NC + c
    base = wid * EPW
    pltpu.sync_copy(src_h.at[pl.ds(base, EPW)], srcv)
    pltpu.sync_copy(dst_h.at[pl.ds(base, EPW)], dstv)
    pltpu.sync_copy(vb_h, vbuf)

    cols = [(colq0, colk0), (colq1, colk1)]
    pltpu.sync_copy(qT_h.at[pl.ds(0, N)], colq0)
    pltpu.sync_copy(kT_h.at[pl.ds(0, N)], colk0)

    for d in range(D):
        cq, ck = cols[d % 2]
        if d > 0:
            pltpu.make_async_copy(qT_h.at[pl.ds(d * N, N)], cq, sem_c).wait()
            pltpu.make_async_copy(kT_h.at[pl.ds(d * N, N)], ck, sem_c).wait()
        if d + 1 < D:
            nq, nk = cols[(d + 1) % 2]
            pltpu.async_copy(qT_h.at[pl.ds((d + 1) * N, N)], nq, sem_c)
            pltpu.async_copy(kT_h.at[pl.ds((d + 1) * N, N)], nk, sem_c)
        vrow = vbuf[d]

        @plsc.parallel_loop(0, NGRP, unroll=4)
        def _(g):
            off = g * 16
            s16 = srcv[pl.ds(off, 16)]
            d16 = dstv[pl.ds(off, 16)]
            qv = plsc.load_gather(cq, [s16])
            kv = plsc.load_gather(ck, [d16])
            z = qv + kv
            e2 = jnp.exp(z + z)
            t = 1.0 - 2.0 / (e2 + 1.0)
            if d == 0:
                acc[pl.ds(off, 16)] = vrow * t
            else:
                acc[pl.ds(off, 16)] = acc[pl.ds(off, 16)] + vrow * t

    @plsc.parallel_loop(0, NGRP, unroll=4)
    def _(g):
        off = g * 16
        acc[pl.ds(off, 16)] = jnp.exp(acc[pl.ds(off, 16)])

    pltpu.sync_copy(acc, w_h.at[pl.ds(base, EPW)])


def _score(qT, kT, src, dst, vb):
    fn = pl.kernel(
        _score_body,
        out_type=jax.ShapeDtypeStruct((E,), jnp.float32),
        mesh=_MESH,
        compiler_params=_SC_PARAMS,
        scratch_types=[
            pltpu.VMEM((N,), jnp.float32),
            pltpu.VMEM((N,), jnp.float32),
            pltpu.VMEM((N,), jnp.float32),
            pltpu.VMEM((N,), jnp.float32),
            pltpu.VMEM((EPW,), jnp.int32),
            pltpu.VMEM((EPW,), jnp.int32),
            pltpu.VMEM((EPW,), jnp.float32),
            pltpu.VMEM((D, 16), jnp.float32),
            pltpu.SemaphoreType.DMA,
        ],
    )
    return fn(qT, kT, src, dst, vb)


def _agg_body(x_h, src_h, dst_h, w_h, aggp_h, denp_h,
              mb_s, mb_d, mb_w, zbuf, xbufs, agg_sh, den_sh,
              sem_g, sem_s, sem_m):
    c = lax.axis_index("c")
    s = lax.axis_index("s")
    wid = s * NC + c
    base = wid * EPW

    def _splat(v):
        return jnp.broadcast_to(v, (16,)).astype(jnp.int32)

    for r in range(CH):
        for h in range(F // 16):
            xbufs[0, r, pl.ds(h * 16, 16)] = jnp.zeros((16,), jnp.float32)
    for h in range(CH // 16):
        zbuf[pl.ds(h * 16, 16)] = jnp.zeros((16,), jnp.float32)
    for i in range((NZCH + NS - 1) // NS):
        j = s + i * NS

        @pl.when(j < NZCH)
        def _():
            pltpu.sync_copy(xbufs.at[0], agg_sh.at[pl.ds(j * CH, CH)])
            pltpu.sync_copy(zbuf, den_sh.at[pl.ds(j * CH, CH)])
    plsc.subcore_barrier()

    def _meta_slices(j):
        return (src_h.at[pl.ds(base + j * CH, CH)],
                dst_h.at[pl.ds(base + j * CH, CH)],
                w_h.at[pl.ds(base + j * CH, CH)])

    s0, d0, w0 = _meta_slices(0)
    pltpu.sync_copy(s0, mb_s.at[0])
    pltpu.sync_copy(d0, mb_d.at[0])
    pltpu.sync_copy(w0, mb_w.at[0])
    s1, d1, w1 = _meta_slices(1)
    pltpu.async_copy(s1, mb_s.at[1], sem_m)
    pltpu.async_copy(d1, mb_d.at[1], sem_m)
    pltpu.async_copy(w1, mb_w.at[1], sem_m)
    pltpu.async_copy(x_h.at[mb_s.at[0]], xbufs.at[0], sem_g)

    def chunk_body(j, carry):
        m0 = j % 3
        m1 = (j + 1) % 3
        m2 = (j + 2) % 3

        pltpu.make_async_copy(x_h.at[mb_s.at[m0]], xbufs.at[m0], sem_g).wait()

        @plsc.parallel_loop(0, CH, unroll=2)
        def _(e):
            a16 = plsc.load_gather(mb_w, [_splat(m0), _splat(e)])
            for h in range(F // 16):
                xbufs[m0, e, pl.ds(h * 16, 16)] = (
                    a16 * xbufs[m0, e, pl.ds(h * 16, 16)])

        @pl.when(j > 0)
        def _():
            pltpu.make_async_copy(
                xbufs.at[m2], agg_sh.at[mb_d.at[m2]], sem_s).wait()
            pltpu.make_async_copy(
                mb_w.at[m2], den_sh.at[mb_d.at[m2]], sem_s).wait()

        pltpu.async_copy(xbufs.at[m0], agg_sh.at[mb_d.at[m0]], sem_s,
                         add=True)
        pltpu.async_copy(mb_w.at[m0], den_sh.at[mb_d.at[m0]], sem_s,
                         add=True)

        @pl.when(j + 1 < NCH)
        def _():
            sj, dj, wj = _meta_slices(j + 1)
            pltpu.make_async_copy(sj, mb_s.at[m1], sem_m).wait()
            pltpu.make_async_copy(dj, mb_d.at[m1], sem_m).wait()
            pltpu.make_async_copy(wj, mb_w.at[m1], sem_m).wait()

        @pl.when(j + 2 < NCH)
        def _():
            sj, dj, wj = _meta_slices(j + 2)
            pltpu.async_copy(sj, mb_s.at[m2], sem_m)
            pltpu.async_copy(dj, mb_d.at[m2], sem_m)
            pltpu.async_copy(wj, mb_w.at[m2], sem_m)

        @pl.when(j + 1 < NCH)
        def _():
            pltpu.async_copy(x_h.at[mb_s.at[m1]], xbufs.at[m1], sem_g)
        return carry
    lax.fori_loop(0, NCH, chunk_body, 0)

    ml = (NCH - 1) % 3
    pltpu.make_async_copy(xbufs.at[ml], agg_sh.at[mb_d.at[ml]], sem_s).wait()
    pltpu.make_async_copy(mb_w.at[ml], den_sh.at[mb_d.at[ml]], sem_s).wait()

    plsc.subcore_barrier()
    for i in range((NZCH + NS - 1) // NS):
        j = s + i * NS

        @pl.when(j < NZCH)
        def _():
            pltpu.sync_copy(agg_sh.at[pl.ds(j * CH, CH)],
                            aggp_h.at[c, pl.ds(j * CH, CH)])
            pltpu.sync_copy(den_sh.at[pl.ds(j * CH, CH)], zbuf)
            pltpu.sync_copy(zbuf, denp_h.at[pl.ds(c * N + j * CH, CH)])


def _agg(x2d, src, dst, w):
    fn = pl.kernel(
        _agg_body,
        out_type=(jax.ShapeDtypeStruct((NC, N, F), jnp.float32),
                  jax.ShapeDtypeStruct((NC * N,), jnp.float32)),
        mesh=_MESH,
        compiler_params=_SC_PARAMS,
        scratch_types=[
            pltpu.VMEM((3, CH), jnp.int32),
            pltpu.VMEM((3, CH), jnp.int32),
            pltpu.VMEM((3, CH), jnp.float32),
            pltpu.VMEM((CH,), jnp.float32),
            pltpu.VMEM((3, CH, F), jnp.float32),
            pltpu.VMEM_SHARED((N, F), jnp.float32),
            pltpu.VMEM_SHARED((N,), jnp.float32),
            pltpu.SemaphoreType.DMA,
            pltpu.SemaphoreType.DMA,
            pltpu.SemaphoreType.DMA,
        ],
    )
    return fn(x2d, src, dst, w)


def _fc_body(x_ref, a0_ref, a1_ref, d0_ref, d1_ref, w1_ref, w2_ref, b_ref,
             o_ref):
    inv = 1.0 / (d0_ref[...] + d1_ref[...] + 1e-8)
    ag = (a0_ref[...] + a1_ref[...]) * inv
    acc = jnp.dot(x_ref[...], w1_ref[...], preferred_element_type=jnp.float32)
    acc = acc + jnp.dot(ag, w2_ref[...], preferred_element_type=jnp.float32)
    o_ref[...] = jnp.maximum(acc + b_ref[...], 0.0)


def _fc(x2d, a0, a1, d0, d1, w1T, w2T, b2d):
    BLK = 1000
    return pl.pallas_call(
        _fc_body,
        grid=(N // BLK,),
        in_specs=[
            pl.BlockSpec((BLK, F), lambda i: (i, 0)),
            pl.BlockSpec((BLK, F), lambda i: (i, 0)),
            pl.BlockSpec((BLK, F), lambda i: (i, 0)),
            pl.BlockSpec((BLK, 1), lambda i: (i, 0)),
            pl.BlockSpec((BLK, 1), lambda i: (i, 0)),
            pl.BlockSpec((F, OUT), lambda i: (0, 0)),
            pl.BlockSpec((F, OUT), lambda i: (0, 0)),
            pl.BlockSpec((1, OUT), lambda i: (0, 0)),
        ],
        out_specs=pl.BlockSpec((BLK, OUT), lambda i: (i, 0)),
        out_shape=jax.ShapeDtypeStruct((N, OUT), jnp.float32),
    )(x2d, a0, a1, d0, d1, w1T, w2T, b2d)


def kernel(x, embeddings, edge_index, Wq, Wk, v_w, fc_w, fc_b):
    x2d = x[0]
    embT = embeddings.T
    src = edge_index[0]
    dst = edge_index[1]
    vb = jnp.broadcast_to(v_w.reshape(D, 1), (D, 16))

    qT, kT = _qk(embT, Wq, Wk)
    w = _score(qT.reshape(D * N), kT.reshape(D * N), src, dst, vb)

    aggp, denp = _agg(x2d, src, dst, w)
    denp2 = denp.reshape(NC, N)

    out2d = _fc(x2d, aggp[0], aggp[1],
                denp2[0].reshape(N, 1), denp2[1].reshape(N, 1),
                fc_w[:, :F].T, fc_w[:, F:].T, fc_b.reshape(1, OUT))
    return out2d.reshape(1, N, OUT)


# speedup vs baseline: 23.9016x; 1.1875x over previous
import jax
import jax.numpy as jnp
from jax import lax
from jax.experimental import pallas as pl
from jax.experimental.pallas import tpu as pltpu
from jax.experimental.pallas import tpu_sc as plsc

N = 10000
E = 320000
F = 128
D = 16
OUT = 128
NC = 2
NS = 16
NW = NC * NS
EPW = E // NW
NGRP = EPW // 16
CH = 80
NCH = EPW // CH
NZCH = N // CH

_MESH = plsc.VectorSubcoreMesh(
    core_axis_name="c", subcore_axis_name="s", num_cores=NC, num_subcores=NS)
_SC_PARAMS = pltpu.CompilerParams(needs_layout_passes=False)


def _qk_body(embT_ref, wq_ref, wk_ref, qT_ref, kT_ref):
    embT = embT_ref[...]
    qT_ref[...] = jnp.dot(wq_ref[...], embT, preferred_element_type=jnp.float32)
    kT_ref[...] = jnp.dot(wk_ref[...], embT, preferred_element_type=jnp.float32)


def _qk(embT, Wq, Wk):
    return pl.pallas_call(
        _qk_body,
        out_shape=(jax.ShapeDtypeStruct((D, N), jnp.float32),
                   jax.ShapeDtypeStruct((D, N), jnp.float32)),
    )(embT, Wq, Wk)


def _score_body(qT_h, kT_h, src_h, dst_h, vb_h, w_h,
                colq0, colk0, colq1, colk1, srcv, dstv, acc, vbuf, sem_c):
    c = lax.axis_index("c")
    s = lax.axis_index("s")
    wid = s * NC + c
    base = wid * EPW
    pltpu.sync_copy(src_h.at[pl.ds(base, EPW)], srcv)
    pltpu.sync_copy(dst_h.at[pl.ds(base, EPW)], dstv)
    pltpu.sync_copy(vb_h, vbuf)

    cols = [(colq0, colk0), (colq1, colk1)]
    pltpu.sync_copy(qT_h.at[pl.ds(0, N)], colq0)
    pltpu.sync_copy(kT_h.at[pl.ds(0, N)], colk0)

    for d in range(D):
        cq, ck = cols[d % 2]
        if d > 0:
            pltpu.make_async_copy(qT_h.at[pl.ds(d * N, N)], cq, sem_c).wait()
            pltpu.make_async_copy(kT_h.at[pl.ds(d * N, N)], ck, sem_c).wait()
        if d + 1 < D:
            nq, nk = cols[(d + 1) % 2]
            pltpu.async_copy(qT_h.at[pl.ds((d + 1) * N, N)], nq, sem_c)
            pltpu.async_copy(kT_h.at[pl.ds((d + 1) * N, N)], nk, sem_c)
        vrow = vbuf[d]

        @plsc.parallel_loop(0, NGRP, unroll=4)
        def _(g):
            off = g * 16
            s16 = srcv[pl.ds(off, 16)]
            d16 = dstv[pl.ds(off, 16)]
            qv = plsc.load_gather(cq, [s16])
            kv = plsc.load_gather(ck, [d16])
            z = qv + kv
            e2 = jnp.exp(z + z)
            t = 1.0 - 2.0 / (e2 + 1.0)
            if d == 0:
                acc[pl.ds(off, 16)] = vrow * t
            else:
                acc[pl.ds(off, 16)] = acc[pl.ds(off, 16)] + vrow * t

    @plsc.parallel_loop(0, NGRP, unroll=4)
    def _(g):
        off = g * 16
        acc[pl.ds(off, 16)] = jnp.exp(acc[pl.ds(off, 16)])

    pltpu.sync_copy(acc, w_h.at[pl.ds(base, EPW)])


def _score(qT, kT, src, dst, vb):
    fn = pl.kernel(
        _score_body,
        out_type=jax.ShapeDtypeStruct((E,), jnp.float32),
        mesh=_MESH,
        compiler_params=_SC_PARAMS,
        scratch_types=[
            pltpu.VMEM((N,), jnp.float32),
            pltpu.VMEM((N,), jnp.float32),
            pltpu.VMEM((N,), jnp.float32),
            pltpu.VMEM((N,), jnp.float32),
            pltpu.VMEM((EPW,), jnp.int32),
            pltpu.VMEM((EPW,), jnp.int32),
            pltpu.VMEM((EPW,), jnp.float32),
            pltpu.VMEM((D, 16), jnp.float32),
            pltpu.SemaphoreType.DMA,
        ],
    )
    return fn(qT, kT, src, dst, vb)


def _agg_body(x_h, src_h, dst_h, w_h, aggp_h, denp_h,
              mb_s, mb_d, mb_w, zbuf, xbufs, agg_sh, den_sh,
              sem_g, sem_s, sem_m):
    c = lax.axis_index("c")
    s = lax.axis_index("s")
    wid = s * NC + c
    base = wid * EPW

    def _splat(v):
        return jnp.broadcast_to(v, (16,)).astype(jnp.int32)

    for r in range(CH):
        for h in range(F // 16):
            xbufs[0, r, pl.ds(h * 16, 16)] = jnp.zeros((16,), jnp.float32)
    for h in range(CH // 16):
        zbuf[pl.ds(h * 16, 16)] = jnp.zeros((16,), jnp.float32)
    for i in range((NZCH + NS - 1) // NS):
        j = s + i * NS

        @pl.when(j < NZCH)
        def _():
            pltpu.sync_copy(xbufs.at[0], agg_sh.at[pl.ds(j * CH, CH)])
            pltpu.sync_copy(zbuf, den_sh.at[pl.ds(j * CH, CH)])
    plsc.subcore_barrier()

    def _meta_slices(j):
        return (src_h.at[pl.ds(base + j * CH, CH)],
                dst_h.at[pl.ds(base + j * CH, CH)],
                w_h.at[pl.ds(base + j * CH, CH)])

    s0, d0, w0 = _meta_slices(0)
    pltpu.sync_copy(s0, mb_s.at[0])
    pltpu.sync_copy(d0, mb_d.at[0])
    pltpu.sync_copy(w0, mb_w.at[0])
    s1, d1, w1 = _meta_slices(1)
    pltpu.async_copy(s1, mb_s.at[1], sem_m)
    pltpu.async_copy(d1, mb_d.at[1], sem_m)
    pltpu.async_copy(w1, mb_w.at[1], sem_m)
    pltpu.async_copy(x_h.at[mb_s.at[0]], xbufs.at[0], sem_g)

    def chunk_body(j, carry):
        m0 = j % 3
        m1 = (j + 1) % 3
        m2 = (j + 2) % 3

        pltpu.make_async_copy(x_h.at[mb_s.at[m0]], xbufs.at[m0], sem_g).wait()

        @pl.when(j + 1 < NCH)
        def _():
            sj, dj, wj = _meta_slices(j + 1)
            pltpu.make_async_copy(sj, mb_s.at[m1], sem_m).wait()
            pltpu.make_async_copy(dj, mb_d.at[m1], sem_m).wait()
            pltpu.make_async_copy(wj, mb_w.at[m1], sem_m).wait()
            pltpu.async_copy(x_h.at[mb_s.at[m1]], xbufs.at[m1], sem_g)

        @plsc.parallel_loop(0, CH, unroll=2)
        def _(e):
            a16 = plsc.load_gather(mb_w, [_splat(m0), _splat(e)])
            for h in range(F // 16):
                xbufs[m0, e, pl.ds(h * 16, 16)] = (
                    a16 * xbufs[m0, e, pl.ds(h * 16, 16)])

        @pl.when(j > 0)
        def _():
            pltpu.make_async_copy(
                xbufs.at[m2], agg_sh.at[mb_d.at[m2]], sem_s).wait()
            pltpu.make_async_copy(
                mb_w.at[m2], den_sh.at[mb_d.at[m2]], sem_s).wait()

        pltpu.async_copy(xbufs.at[m0], agg_sh.at[mb_d.at[m0]], sem_s,
                         add=True)
        pltpu.async_copy(mb_w.at[m0], den_sh.at[mb_d.at[m0]], sem_s,
                         add=True)

        @pl.when(j + 2 < NCH)
        def _():
            sj, dj, wj = _meta_slices(j + 2)
            pltpu.async_copy(sj, mb_s.at[m2], sem_m)
            pltpu.async_copy(dj, mb_d.at[m2], sem_m)
            pltpu.async_copy(wj, mb_w.at[m2], sem_m)
        return carry
    lax.fori_loop(0, NCH, chunk_body, 0)

    ml = (NCH - 1) % 3
    pltpu.make_async_copy(xbufs.at[ml], agg_sh.at[mb_d.at[ml]], sem_s).wait()
    pltpu.make_async_copy(mb_w.at[ml], den_sh.at[mb_d.at[ml]], sem_s).wait()

    plsc.subcore_barrier()
    for i in range((NZCH + NS - 1) // NS):
        j = s + i * NS

        @pl.when(j < NZCH)
        def _():
            pltpu.sync_copy(agg_sh.at[pl.ds(j * CH, CH)],
                            aggp_h.at[c, pl.ds(j * CH, CH)])
            pltpu.sync_copy(den_sh.at[pl.ds(j * CH, CH)], zbuf)
            pltpu.sync_copy(zbuf, denp_h.at[pl.ds(c * N + j * CH, CH)])


def _agg(x2d, src, dst, w):
    fn = pl.kernel(
        _agg_body,
        out_type=(jax.ShapeDtypeStruct((NC, N, F), jnp.float32),
                  jax.ShapeDtypeStruct((NC * N,), jnp.float32)),
        mesh=_MESH,
        compiler_params=_SC_PARAMS,
        scratch_types=[
            pltpu.VMEM((3, CH), jnp.int32),
            pltpu.VMEM((3, CH), jnp.int32),
            pltpu.VMEM((3, CH), jnp.float32),
            pltpu.VMEM((CH,), jnp.float32),
            pltpu.VMEM((3, CH, F), jnp.float32),
            pltpu.VMEM_SHARED((N, F), jnp.float32),
            pltpu.VMEM_SHARED((N,), jnp.float32),
            pltpu.SemaphoreType.DMA,
            pltpu.SemaphoreType.DMA,
            pltpu.SemaphoreType.DMA,
        ],
    )
    return fn(x2d, src, dst, w)


def _fc_body(x_ref, a0_ref, a1_ref, d0_ref, d1_ref, w1_ref, w2_ref, b_ref,
             o_ref):
    inv = 1.0 / (d0_ref[...] + d1_ref[...] + 1e-8)
    ag = (a0_ref[...] + a1_ref[...]) * inv
    acc = jnp.dot(x_ref[...], w1_ref[...], preferred_element_type=jnp.float32)
    acc = acc + jnp.dot(ag, w2_ref[...], preferred_element_type=jnp.float32)
    o_ref[...] = jnp.maximum(acc + b_ref[...], 0.0)


def _fc(x2d, a0, a1, d0, d1, w1T, w2T, b2d):
    BLK = 1000
    return pl.pallas_call(
        _fc_body,
        grid=(N // BLK,),
        in_specs=[
            pl.BlockSpec((BLK, F), lambda i: (i, 0)),
            pl.BlockSpec((BLK, F), lambda i: (i, 0)),
            pl.BlockSpec((BLK, F), lambda i: (i, 0)),
            pl.BlockSpec((BLK, 1), lambda i: (i, 0)),
            pl.BlockSpec((BLK, 1), lambda i: (i, 0)),
            pl.BlockSpec((F, OUT), lambda i: (0, 0)),
            pl.BlockSpec((F, OUT), lambda i: (0, 0)),
            pl.BlockSpec((1, OUT), lambda i: (0, 0)),
        ],
        out_specs=pl.BlockSpec((BLK, OUT), lambda i: (i, 0)),
        out_shape=jax.ShapeDtypeStruct((N, OUT), jnp.float32),
    )(x2d, a0, a1, d0, d1, w1T, w2T, b2d)


def kernel(x, embeddings, edge_index, Wq, Wk, v_w, fc_w, fc_b):
    x2d = x[0]
    embT = embeddings.T
    src = edge_index[0]
    dst = edge_index[1]
    vb = jnp.broadcast_to(v_w.reshape(D, 1), (D, 16))

    qT, kT = _qk(embT, Wq, Wk)
    w = _score(qT.reshape(D * N), kT.reshape(D * N), src, dst, vb)

    aggp, denp = _agg(x2d, src, dst, w)
    denp2 = denp.reshape(NC, N)

    out2d = _fc(x2d, aggp[0], aggp[1],
                denp2[0].reshape(N, 1), denp2[1].reshape(N, 1),
                fc_w[:, :F].T, fc_w[:, F:].T, fc_b.reshape(1, OUT))
    return out2d.reshape(1, N, OUT)


# speedup vs baseline: 23.9412x; 1.0017x over previous
import jax
import jax.numpy as jnp
from jax import lax
from jax.experimental import pallas as pl
from jax.experimental.pallas import tpu as pltpu
from jax.experimental.pallas import tpu_sc as plsc

N = 10000
E = 320000
F = 128
D = 16
OUT = 128
NC = 2
NS = 16
NW = NC * NS
EPW = E // NW
NGRP = EPW // 16
CH = 80
NCH = EPW // CH
NZCH = N // CH

_MESH = plsc.VectorSubcoreMesh(
    core_axis_name="c", subcore_axis_name="s", num_cores=NC, num_subcores=NS)
_SC_PARAMS = pltpu.CompilerParams(needs_layout_passes=False)


def _qk_body(embT_ref, wq_ref, wk_ref, qT_ref, kT_ref):
    embT = embT_ref[...]
    qT_ref[...] = jnp.dot(wq_ref[...], embT, preferred_element_type=jnp.float32)
    kT_ref[...] = jnp.dot(wk_ref[...], embT, preferred_element_type=jnp.float32)


def _qk(embT, Wq, Wk):
    return pl.pallas_call(
        _qk_body,
        out_shape=(jax.ShapeDtypeStruct((D, N), jnp.float32),
                   jax.ShapeDtypeStruct((D, N), jnp.float32)),
    )(embT, Wq, Wk)


def _score_body(qT_h, kT_h, src_h, dst_h, vb_h, w_h,
                colq0, colk0, colq1, colk1, srcv, dstv, acc, vbuf, sem_c):
    c = lax.axis_index("c")
    s = lax.axis_index("s")
    wid = s * NC + c
    base = wid * EPW
    pltpu.sync_copy(src_h.at[pl.ds(base, EPW)], srcv)
    pltpu.sync_copy(dst_h.at[pl.ds(base, EPW)], dstv)
    pltpu.sync_copy(vb_h, vbuf)

    cols = [(colq0, colk0), (colq1, colk1)]
    pltpu.sync_copy(qT_h.at[pl.ds(0, N)], colq0)
    pltpu.sync_copy(kT_h.at[pl.ds(0, N)], colk0)

    for d in range(D):
        cq, ck = cols[d % 2]
        if d > 0:
            pltpu.make_async_copy(qT_h.at[pl.ds(d * N, N)], cq, sem_c).wait()
            pltpu.make_async_copy(kT_h.at[pl.ds(d * N, N)], ck, sem_c).wait()
        if d + 1 < D:
            nq, nk = cols[(d + 1) % 2]
            pltpu.async_copy(qT_h.at[pl.ds((d + 1) * N, N)], nq, sem_c)
            pltpu.async_copy(kT_h.at[pl.ds((d + 1) * N, N)], nk, sem_c)
        vrow = vbuf[d]

        @plsc.parallel_loop(0, NGRP, unroll=4)
        def _(g):
            off = g * 16
            s16 = srcv[pl.ds(off, 16)]
            d16 = dstv[pl.ds(off, 16)]
            qv = plsc.load_gather(cq, [s16])
            kv = plsc.load_gather(ck, [d16])
            z = qv + kv
            e2 = jnp.exp(z + z)
            t = 1.0 - 2.0 / (e2 + 1.0)
            if d == 0:
                acc[pl.ds(off, 16)] = vrow * t
            else:
                acc[pl.ds(off, 16)] = acc[pl.ds(off, 16)] + vrow * t

    @plsc.parallel_loop(0, NGRP, unroll=4)
    def _(g):
        off = g * 16
        acc[pl.ds(off, 16)] = jnp.exp(acc[pl.ds(off, 16)])

    pltpu.sync_copy(acc, w_h.at[pl.ds(base, EPW)])


def _score(qT, kT, src, dst, vb):
    fn = pl.kernel(
        _score_body,
        out_type=jax.ShapeDtypeStruct((E,), jnp.float32),
        mesh=_MESH,
        compiler_params=_SC_PARAMS,
        scratch_types=[
            pltpu.VMEM((N,), jnp.float32),
            pltpu.VMEM((N,), jnp.float32),
            pltpu.VMEM((N,), jnp.float32),
            pltpu.VMEM((N,), jnp.float32),
            pltpu.VMEM((EPW,), jnp.int32),
            pltpu.VMEM((EPW,), jnp.int32),
            pltpu.VMEM((EPW,), jnp.float32),
            pltpu.VMEM((D, 16), jnp.float32),
            pltpu.SemaphoreType.DMA,
        ],
    )
    return fn(qT, kT, src, dst, vb)


def _agg_body(x_h, src_h, dst_h, w_h, aggp_h, denp_h,
              mb_s, mb_d, mb_w, zbuf, xbufs, agg_sh, den_sh,
              sem_g, sem_s, sem_m):
    c = lax.axis_index("c")
    s = lax.axis_index("s")
    wid = s * NC + c
    base = wid * EPW

    def _splat(v):
        return jnp.broadcast_to(v, (16,)).astype(jnp.int32)

    for r in range(CH):
        for h in range(F // 16):
            xbufs[0, r, pl.ds(h * 16, 16)] = jnp.zeros((16,), jnp.float32)
    for h in range(CH // 16):
        zbuf[pl.ds(h * 16, 16)] = jnp.zeros((16,), jnp.float32)
    for i in range((NZCH + NS - 1) // NS):
        j = s + i * NS

        @pl.when(j < NZCH)
        def _():
            pltpu.sync_copy(xbufs.at[0], agg_sh.at[pl.ds(j * CH, CH)])
            pltpu.sync_copy(zbuf, den_sh.at[pl.ds(j * CH, CH)])
    plsc.subcore_barrier()

    def _meta_slices(j):
        return (src_h.at[pl.ds(base + j * CH, CH)],
                dst_h.at[pl.ds(base + j * CH, CH)],
                w_h.at[pl.ds(base + j * CH, CH)])

    s0, d0, w0 = _meta_slices(0)
    pltpu.sync_copy(s0, mb_s.at[0])
    pltpu.sync_copy(d0, mb_d.at[0])
    pltpu.sync_copy(w0, mb_w.at[0])
    s1, d1, w1 = _meta_slices(1)
    pltpu.async_copy(s1, mb_s.at[1], sem_m)
    pltpu.async_copy(d1, mb_d.at[1], sem_m)
    pltpu.async_copy(w1, mb_w.at[1], sem_m)
    pltpu.async_copy(x_h.at[mb_s.at[0]], xbufs.at[0], sem_g)

    def chunk_body(j, carry):
        m0 = j % 3
        m1 = (j + 1) % 3
        m2 = (j + 2) % 3

        pltpu.make_async_copy(x_h.at[mb_s.at[m0]], xbufs.at[m0], sem_g).wait()

        @pl.when(j + 1 < NCH)
        def _():
            sj, dj, wj = _meta_slices(j + 1)
            pltpu.make_async_copy(sj, mb_s.at[m1], sem_m).wait()
            pltpu.make_async_copy(dj, mb_d.at[m1], sem_m).wait()
            pltpu.make_async_copy(wj, mb_w.at[m1], sem_m).wait()
            pltpu.async_copy(x_h.at[mb_s.at[m1]], xbufs.at[m1], sem_g)

        @plsc.parallel_loop(0, CH, unroll=4)
        def _(e):
            a16 = plsc.load_gather(mb_w, [_splat(m0), _splat(e)])
            for h in range(F // 16):
                xbufs[m0, e, pl.ds(h * 16, 16)] = (
                    a16 * xbufs[m0, e, pl.ds(h * 16, 16)])

        @pl.when(j > 0)
        def _():
            pltpu.make_async_copy(
                xbufs.at[m2], agg_sh.at[mb_d.at[m2]], sem_s).wait()
            pltpu.make_async_copy(
                mb_w.at[m2], den_sh.at[mb_d.at[m2]], sem_s).wait()

        pltpu.async_copy(xbufs.at[m0], agg_sh.at[mb_d.at[m0]], sem_s,
                         add=True)
        pltpu.async_copy(mb_w.at[m0], den_sh.at[mb_d.at[m0]], sem_s,
                         add=True)

        @pl.when(j + 2 < NCH)
        def _():
            sj, dj, wj = _meta_slices(j + 2)
            pltpu.async_copy(sj, mb_s.at[m2], sem_m)
            pltpu.async_copy(dj, mb_d.at[m2], sem_m)
            pltpu.async_copy(wj, mb_w.at[m2], sem_m)
        return carry
    lax.fori_loop(0, NCH, chunk_body, 0)

    ml = (NCH - 1) % 3
    pltpu.make_async_copy(xbufs.at[ml], agg_sh.at[mb_d.at[ml]], sem_s).wait()
    pltpu.make_async_copy(mb_w.at[ml], den_sh.at[mb_d.at[ml]], sem_s).wait()

    plsc.subcore_barrier()
    for i in range((NZCH + NS - 1) // NS):
        j = s + i * NS

        @pl.when(j < NZCH)
        def _():
            pltpu.sync_copy(agg_sh.at[pl.ds(j * CH, CH)],
                            aggp_h.at[c, pl.ds(j * CH, CH)])
            pltpu.sync_copy(den_sh.at[pl.ds(j * CH, CH)], zbuf)
            pltpu.sync_copy(zbuf, denp_h.at[pl.ds(c * N + j * CH, CH)])


def _agg(x2d, src, dst, w):
    fn = pl.kernel(
        _agg_body,
        out_type=(jax.ShapeDtypeStruct((NC, N, F), jnp.float32),
                  jax.ShapeDtypeStruct((NC * N,), jnp.float32)),
        mesh=_MESH,
        compiler_params=_SC_PARAMS,
        scratch_types=[
            pltpu.VMEM((3, CH), jnp.int32),
            pltpu.VMEM((3, CH), jnp.int32),
            pltpu.VMEM((3, CH), jnp.float32),
            pltpu.VMEM((CH,), jnp.float32),
            pltpu.VMEM((3, CH, F), jnp.float32),
            pltpu.VMEM_SHARED((N, F), jnp.float32),
            pltpu.VMEM_SHARED((N,), jnp.float32),
            pltpu.SemaphoreType.DMA,
            pltpu.SemaphoreType.DMA,
            pltpu.SemaphoreType.DMA,
        ],
    )
    return fn(x2d, src, dst, w)


def _fc_body(x_ref, a0_ref, a1_ref, d0_ref, d1_ref, w1_ref, w2_ref, b_ref,
             o_ref):
    inv = 1.0 / (d0_ref[...] + d1_ref[...] + 1e-8)
    ag = (a0_ref[...] + a1_ref[...]) * inv
    acc = jnp.dot(x_ref[...], w1_ref[...], preferred_element_type=jnp.float32)
    acc = acc + jnp.dot(ag, w2_ref[...], preferred_element_type=jnp.float32)
    o_ref[...] = jnp.maximum(acc + b_ref[...], 0.0)


def _fc(x2d, a0, a1, d0, d1, w1T, w2T, b2d):
    BLK = 1000
    return pl.pallas_call(
        _fc_body,
        grid=(N // BLK,),
        in_specs=[
            pl.BlockSpec((BLK, F), lambda i: (i, 0)),
            pl.BlockSpec((BLK, F), lambda i: (i, 0)),
            pl.BlockSpec((BLK, F), lambda i: (i, 0)),
            pl.BlockSpec((BLK, 1), lambda i: (i, 0)),
            pl.BlockSpec((BLK, 1), lambda i: (i, 0)),
            pl.BlockSpec((F, OUT), lambda i: (0, 0)),
            pl.BlockSpec((F, OUT), lambda i: (0, 0)),
            pl.BlockSpec((1, OUT), lambda i: (0, 0)),
        ],
        out_specs=pl.BlockSpec((BLK, OUT), lambda i: (i, 0)),
        out_shape=jax.ShapeDtypeStruct((N, OUT), jnp.float32),
    )(x2d, a0, a1, d0, d1, w1T, w2T, b2d)


def kernel(x, embeddings, edge_index, Wq, Wk, v_w, fc_w, fc_b):
    x2d = x[0]
    embT = embeddings.T
    src = edge_index[0]
    dst = edge_index[1]
    vb = jnp.broadcast_to(v_w.reshape(D, 1), (D, 16))

    qT, kT = _qk(embT, Wq, Wk)
    w = _score(qT.reshape(D * N), kT.reshape(D * N), src, dst, vb)

    aggp, denp = _agg(x2d, src, dst, w)
    denp2 = denp.reshape(NC, N)

    out2d = _fc(x2d, aggp[0], aggp[1],
                denp2[0].reshape(N, 1), denp2[1].reshape(N, 1),
                fc_w[:, :F].T, fc_w[:, F:].T, fc_b.reshape(1, OUT))
    return out2d.reshape(1, N, OUT)
